# Initial kernel scaffold; baseline (speedup 1.0000x reference)
import jax
import jax.numpy as jnp
from jax.experimental import pallas as pl


def kernel(x, embed_weight):
    raise NotImplementedError("write your pallas kernel here")


# speedup vs baseline: 1.6943x; 1.6943x over previous
import jax
import jax.numpy as jnp
from jax.experimental import pallas as pl

BLOCK_S = 512


def _add_kernel(x_ref, e_ref, o_ref):
    o_ref[...] = x_ref[...] + e_ref[...][None, :, :]


def kernel(x, embed_weight):
    B, S, D = x.shape
    grid = (S // BLOCK_S, B)
    return pl.pallas_call(
        _add_kernel,
        grid=grid,
        in_specs=[
            pl.BlockSpec((1, BLOCK_S, D), lambda s, b: (b, s, 0)),
            pl.BlockSpec((BLOCK_S, D), lambda s, b: (s, 0)),
        ],
        out_specs=pl.BlockSpec((1, BLOCK_S, D), lambda s, b: (b, s, 0)),
        out_shape=jax.ShapeDtypeStruct((B, S, D), x.dtype),
    )(x, embed_weight)


# speedup vs baseline: 1.8790x; 1.1090x over previous
import jax
import jax.numpy as jnp
from jax.experimental import pallas as pl

BLOCK_S = 1024


def _add_kernel(x_ref, e_ref, o_ref):
    o_ref[...] = x_ref[...] + e_ref[...][None, :, :]


def kernel(x, embed_weight):
    B, S, D = x.shape
    grid = (S // BLOCK_S, B)
    return pl.pallas_call(
        _add_kernel,
        grid=grid,
        in_specs=[
            pl.BlockSpec((1, BLOCK_S, D), lambda s, b: (b, s, 0)),
            pl.BlockSpec((BLOCK_S, D), lambda s, b: (s, 0)),
        ],
        out_specs=pl.BlockSpec((1, BLOCK_S, D), lambda s, b: (b, s, 0)),
        out_shape=jax.ShapeDtypeStruct((B, S, D), x.dtype),
    )(x, embed_weight)


# speedup vs baseline: 1.9896x; 1.0589x over previous
import jax
import jax.numpy as jnp
from jax.experimental import pallas as pl

BLOCK_S = 2048


def _add_kernel(x_ref, e_ref, o_ref):
    o_ref[...] = x_ref[...] + e_ref[...][None, :, :]


def kernel(x, embed_weight):
    B, S, D = x.shape
    grid = (S // BLOCK_S, B)
    return pl.pallas_call(
        _add_kernel,
        grid=grid,
        in_specs=[
            pl.BlockSpec((1, BLOCK_S, D), lambda s, b: (b, s, 0)),
            pl.BlockSpec((BLOCK_S, D), lambda s, b: (s, 0)),
        ],
        out_specs=pl.BlockSpec((1, BLOCK_S, D), lambda s, b: (b, s, 0)),
        out_shape=jax.ShapeDtypeStruct((B, S, D), x.dtype),
    )(x, embed_weight)


# speedup vs baseline: 1.9964x; 1.0034x over previous
import jax
import jax.numpy as jnp
from jax.experimental import pallas as pl
from jax.experimental.pallas import tpu as pltpu

BLOCK_S = 2048


def _add_kernel(x_ref, e_ref, o_ref):
    o_ref[...] = x_ref[...] + e_ref[...][None, :, :]


def kernel(x, embed_weight):
    B, S, D = x.shape
    grid = (S // BLOCK_S, B)
    return pl.pallas_call(
        _add_kernel,
        grid=grid,
        in_specs=[
            pl.BlockSpec((1, BLOCK_S, D), lambda s, b: (b, s, 0)),
            pl.BlockSpec((BLOCK_S, D), lambda s, b: (s, 0)),
        ],
        out_specs=pl.BlockSpec((1, BLOCK_S, D), lambda s, b: (b, s, 0)),
        out_shape=jax.ShapeDtypeStruct((B, S, D), x.dtype),
        compiler_params=pltpu.CompilerParams(
            dimension_semantics=("parallel", "parallel")
        ),
    )(x, embed_weight)
